# baseline (speedup 1.0000x reference)
import functools
from math import comb

import jax
import jax.numpy as jnp
import numpy as np
from jax import lax
from jax.experimental import pallas as pl
from jax.experimental.pallas import tpu as pltpu
from jax.experimental.pallas import tpu_sc as plsc

N = 10000
E = 320000
K = 10
CH = 64
NC, NS = 2, 16
CHH = CH // NC
NP = 10240
NPT = NP // NS
ECH = 128
EPT = 20480
NCHUNK = EPT // ECH
CPB = 16
NBLK = NCHUNK // CPB
EP = NS * EPT
RB = 1000

_CM = np.zeros((K + 1, K + 1), np.float64)
for _j in range(K + 1):
    for _i in range(_j + 1):
        _CM[_j, _i] = (2.0 ** -_j) * comb(K, _j) * ((-1) ** (_j - _i)) * comb(_j, _i)
_CM = _CM.astype(np.float32)


def _mlp_body(x_ref, w0_ref, b0_ref, w1_ref, b1_ref, o_ref):
    hh = jnp.dot(
        x_ref[...], w0_ref[...],
        preferred_element_type=jnp.float32, precision=lax.Precision.HIGHEST,
    )
    hh = jnp.maximum(hh + b0_ref[...], 0.0)
    o_ref[...] = (
        jnp.dot(
            hh, w1_ref[...],
            preferred_element_type=jnp.float32, precision=lax.Precision.HIGHEST,
        )
        + b1_ref[...]
    )


_mlp = pl.pallas_call(
    _mlp_body,
    grid=(N // RB,),
    in_specs=[
        pl.BlockSpec((RB, 128), lambda i: (i, 0)),
        pl.BlockSpec((128, 128), lambda i: (0, 0)),
        pl.BlockSpec((1, 128), lambda i: (0, 0)),
        pl.BlockSpec((128, CH), lambda i: (0, 0)),
        pl.BlockSpec((1, CH), lambda i: (0, 0)),
    ],
    out_specs=pl.BlockSpec((RB, CH), lambda i: (i, 0)),
    out_shape=jax.ShapeDtypeStruct((N, CH), jnp.float32),
)


def _lsm_body(y_ref, o_ref):
    y = y_ref[...]
    m = jnp.max(y, axis=1, keepdims=True)
    sh = y - m
    ssum = jnp.sum(jnp.exp(sh), axis=1, keepdims=True)
    o_ref[...] = sh - jnp.log(ssum)


_lsm = pl.pallas_call(
    _lsm_body,
    grid=(N // RB,),
    in_specs=[pl.BlockSpec((RB, CH), lambda i: (i, 0))],
    out_specs=pl.BlockSpec((RB, CH), lambda i: (i, 0)),
    out_shape=jax.ShapeDtypeStruct((N, CH), jnp.float32),
)


_mesh = plsc.VectorSubcoreMesh(
    core_axis_name="c", subcore_axis_name="s", num_cores=NC, num_subcores=NS
)


def _splat(val):
    return jnp.full((16,), val, jnp.int32)


NBUF = 6
LEAD = 3


@functools.partial(
    pl.kernel,
    out_type=(
        jax.ShapeDtypeStruct((NC, NP, CHH), jnp.float32),
        jax.ShapeDtypeStruct((NC, NP, CHH), jnp.float32),
    ),
    mesh=_mesh,
    scratch_types=[
        pltpu.VMEM_SHARED((NP, CHH), jnp.float32),
        pltpu.VMEM_SHARED((NP,), jnp.float32),
        pltpu.VMEM((NPT, CHH), jnp.float32),
        pltpu.VMEM((NPT, CHH), jnp.float32),
        pltpu.VMEM((NPT, CHH), jnp.float32),
        pltpu.VMEM((ECH, CHH), jnp.float32),
        pltpu.VMEM((ECH, CHH), jnp.float32),
        pltpu.VMEM((ECH, CHH), jnp.float32),
        pltpu.VMEM((ECH, CHH), jnp.float32),
        pltpu.VMEM((ECH, CHH), jnp.float32),
        pltpu.VMEM((ECH, CHH), jnp.float32),
        pltpu.VMEM((CPB, 2, ECH), jnp.int32),
        pltpu.VMEM((CPB, 2, ECH), jnp.int32),
        pltpu.VMEM((NPT,), jnp.float32),
        pltpu.VMEM((ECH,), jnp.float32),
        pltpu.VMEM((16,), jnp.float32),
    ]
    + [pltpu.SemaphoreType.DMA] * (2 * NBUF + 2),
    compiler_params=pltpu.CompilerParams(
        needs_layout_passes=False, use_tc_tiling_on_sc=False
    ),
)
def _bern_sc(idx_hbm, h_hbm, coef_hbm, ones_hbm, zrow_hbm, zcol_hbm,
             y_out, z_hbm,
             g_sh, deg_sh, y_v, h_v, gbuf,
             r0, r1, r2, r3, r4, r5,
             idxblk0, idxblk1, dinv_v, ones_v, coef_v,
             *sems):
    rb = (r0, r1, r2, r3, r4, r5)
    semr = sems[:NBUF]
    semw = sems[NBUF:2 * NBUF]
    semi = sems[2 * NBUF:]
    idxb = (idxblk0, idxblk1)
    c = lax.axis_index("c")
    s = lax.axis_index("s")
    base = pl.multiple_of(s * NPT, NPT)
    zview = z_hbm.at[c]

    def _zero_g_slice():
        for q in range(NPT // ECH):
            pltpu.sync_copy(zrow_hbm, g_sh.at[pl.ds(base + q * ECH, ECH)])

    pltpu.sync_copy(h_hbm.at[c, pl.ds(base, NPT)], h_v)
    pltpu.sync_copy(coef_hbm, coef_v)
    pltpu.sync_copy(ones_hbm, ones_v)
    pltpu.sync_copy(zcol_hbm, dinv_v)
    pltpu.sync_copy(dinv_v, deg_sh.at[pl.ds(base, NPT)])
    plsc.subcore_barrier()

    def _degblk(b, cc):
        pltpu.sync_copy(idx_hbm.at[s, b], idxblk0)

        def _dchunk(j, c2):
            pltpu.sync_copy(ones_v, deg_sh.at[idxblk0.at[j, 0]], add=True)
            return c2

        lax.fori_loop(0, CPB, _dchunk, 0)
        return cc

    lax.fori_loop(0, NBLK, _degblk, 0)
    plsc.subcore_barrier()

    pltpu.sync_copy(deg_sh.at[pl.ds(base, NPT)], dinv_v)

    def _invsqrt(i, carry):
        d = dinv_v[pl.ds(i * 16, 16)]
        xh = d * 0.5
        ib = lax.bitcast_convert_type(d, jnp.int32)
        ib = 0x5F3759DF - lax.shift_right_arithmetic(ib, 1)
        f = lax.bitcast_convert_type(ib, jnp.float32)
        f = f * (1.5 - xh * f * f)
        f = f * (1.5 - xh * f * f)
        f = f * (1.5 - xh * f * f)
        dinv_v[pl.ds(i * 16, 16)] = jnp.where(d > 0.5, f, 0.0)
        return carry

    lax.fori_loop(0, NPT // 16, _invsqrt, 0)

    ak = plsc.load_gather(coef_v, [_splat(K)])

    def _init_node(n, carry):
        dv = plsc.load_gather(dinv_v, [jnp.full((16,), n, jnp.int32)])
        for half in range(2):
            sl = pl.ds(half * 16, 16)
            yv = ak * h_v[n, sl]
            y_v[n, sl] = yv
            gbuf[n, sl] = dv * yv
        return carry

    lax.fori_loop(0, NPT, _init_node, 0)
    pltpu.sync_copy(gbuf, zview.at[pl.ds(base, NPT)])
    _zero_g_slice()
    plsc.subcore_barrier()

    def _edge_sweep():
        gd = [None] * NBUF
        sd = [None] * NBUF
        iw = [None, None]
        pltpu.sync_copy(idx_hbm.at[s, 0], idxblk0)
        for k in range(LEAD):
            bq = k % NBUF
            gd[bq] = pltpu.make_async_copy(
                zview.at[idxblk0.at[k, 0]], rb[bq], semr[bq])
            gd[bq].start()
        for b in range(NBLK):
            cur = idxb[b % 2]
            if b + 1 < NBLK:
                iw[(b + 1) % 2] = pltpu.make_async_copy(
                    idx_hbm.at[s, b + 1], idxb[(b + 1) % 2], semi[(b + 1) % 2])
                iw[(b + 1) % 2].start()
            for q in range(CPB):
                k = b * CPB + q
                kk = k + LEAD
                if kk < NCHUNK:
                    tb, tq = divmod(kk, CPB)
                    nb = kk % NBUF
                    if tb != b and tq == 0:
                        iw[tb % 2].wait()
                        iw[tb % 2] = None
                    if sd[nb] is not None:
                        sd[nb].wait()
                        sd[nb] = None
                    gd[nb] = pltpu.make_async_copy(
                        zview.at[idxb[tb % 2].at[tq, 0]], rb[nb], semr[nb])
                    gd[nb].start()
                bq = k % NBUF
                gd[bq].wait()
                gd[bq] = None
                sd[bq] = pltpu.make_async_copy(
                    rb[bq], g_sh.at[cur.at[q, 1]], semw[bq])
                sd[bq].start(add=True)
        for i in range(NBUF):
            if sd[i] is not None:
                sd[i].wait()

    def _horner(t, carry):
        _edge_sweep()
        plsc.subcore_barrier()

        aj = plsc.load_gather(coef_v, [_splat(0) + (K - 1 - t)])
        pltpu.sync_copy(g_sh.at[pl.ds(base, NPT)], gbuf)
        _zero_g_slice()

        def _comb(n, cc):
            dv = plsc.load_gather(dinv_v, [jnp.full((16,), n, jnp.int32)])
            for half in range(2):
                sl = pl.ds(half * 16, 16)
                yv = y_v[n, sl] - dv * gbuf[n, sl] + aj * h_v[n, sl]
                y_v[n, sl] = yv
                gbuf[n, sl] = dv * yv
            return cc

        lax.fori_loop(0, NPT, _comb, 0)
        pltpu.sync_copy(gbuf, zview.at[pl.ds(base, NPT)])
        plsc.subcore_barrier()
        return carry

    lax.fori_loop(0, K, _horner, 0)
    pltpu.sync_copy(y_v, y_out.at[c, pl.ds(base, NPT)])


def kernel(x, edge_index, W0, b0, W1, b1, temp):
    h = _mlp(x, W0, b0.reshape(1, -1), W1, b1.reshape(1, -1))

    a = jnp.sum(jnp.asarray(_CM) * jax.nn.relu(temp)[None, :], axis=1)
    coef = jnp.zeros((16,), jnp.float32).at[: K + 1].set(a)

    h3 = jnp.pad(h, ((0, NP - N), (0, 0))).reshape(NP, NC, CHH).transpose(1, 0, 2)
    row = edge_index[0].astype(jnp.int32)
    col = edge_index[1].astype(jnp.int32)
    pad = jnp.full((EP - E,), N, jnp.int32)
    rowp = jnp.concatenate([row, pad]).reshape(NS, NBLK, CPB, 1, ECH)
    colp = jnp.concatenate([col, pad]).reshape(NS, NBLK, CPB, 1, ECH)
    idx_all = jnp.concatenate([rowp, colp], axis=3)

    y3, _ = _bern_sc(
        idx_all, h3, coef,
        jnp.ones((ECH,), jnp.float32),
        jnp.zeros((ECH, CHH), jnp.float32),
        jnp.zeros((NPT,), jnp.float32),
    )
    y = y3.transpose(1, 0, 2).reshape(NP, CH)[:N]
    return _lsm(y)


# speedup vs baseline: 2.0027x; 2.0027x over previous
import functools
from math import comb

import jax
import jax.numpy as jnp
import numpy as np
from jax import lax
from jax.experimental import pallas as pl
from jax.experimental.pallas import tpu as pltpu
from jax.experimental.pallas import tpu_sc as plsc

N = 10000
E = 320000
K = 10
CH = 64
NC, NS = 2, 16
CHH = CH // NC
NP = 10240
NPT = NP // NS
ECH = 128
EPT = 20480
NCHUNK = EPT // ECH
CPB = 16
NBLK = NCHUNK // CPB
EP = NS * EPT
RB = 1000

_CM = np.zeros((K + 1, K + 1), np.float64)
for _j in range(K + 1):
    for _i in range(_j + 1):
        _CM[_j, _i] = (2.0 ** -_j) * comb(K, _j) * ((-1) ** (_j - _i)) * comb(_j, _i)
_CM = _CM.astype(np.float32)


def _mlp_body(x_ref, w0_ref, b0_ref, w1_ref, b1_ref, o_ref):
    hh = jnp.dot(
        x_ref[...], w0_ref[...],
        preferred_element_type=jnp.float32, precision=lax.Precision.HIGHEST,
    )
    hh = jnp.maximum(hh + b0_ref[...], 0.0)
    o_ref[...] = (
        jnp.dot(
            hh, w1_ref[...],
            preferred_element_type=jnp.float32, precision=lax.Precision.HIGHEST,
        )
        + b1_ref[...]
    )


_mlp = pl.pallas_call(
    _mlp_body,
    grid=(N // RB,),
    in_specs=[
        pl.BlockSpec((RB, 128), lambda i: (i, 0)),
        pl.BlockSpec((128, 128), lambda i: (0, 0)),
        pl.BlockSpec((1, 128), lambda i: (0, 0)),
        pl.BlockSpec((128, CH), lambda i: (0, 0)),
        pl.BlockSpec((1, CH), lambda i: (0, 0)),
    ],
    out_specs=pl.BlockSpec((RB, CH), lambda i: (i, 0)),
    out_shape=jax.ShapeDtypeStruct((N, CH), jnp.float32),
)


def _lsm_body(y_ref, o_ref):
    y = y_ref[...]
    m = jnp.max(y, axis=1, keepdims=True)
    sh = y - m
    ssum = jnp.sum(jnp.exp(sh), axis=1, keepdims=True)
    o_ref[...] = sh - jnp.log(ssum)


_lsm = pl.pallas_call(
    _lsm_body,
    grid=(N // RB,),
    in_specs=[pl.BlockSpec((RB, CH), lambda i: (i, 0))],
    out_specs=pl.BlockSpec((RB, CH), lambda i: (i, 0)),
    out_shape=jax.ShapeDtypeStruct((N, CH), jnp.float32),
)


_mesh = plsc.VectorSubcoreMesh(
    core_axis_name="c", subcore_axis_name="s", num_cores=NC, num_subcores=NS
)


def _splat(val):
    return jnp.full((16,), val, jnp.int32)


NBUF = 4
LEAD = 2


@functools.partial(
    pl.kernel,
    out_type=jax.ShapeDtypeStruct((NC, NP, CHH), jnp.float32),
    mesh=_mesh,
    scratch_types=[
        pltpu.VMEM_SHARED((NP, CHH), jnp.float32),
        pltpu.VMEM_SHARED((NP, CHH), jnp.float32),
        pltpu.VMEM_SHARED((NP,), jnp.float32),
        pltpu.VMEM((NPT, CHH), jnp.float32),
        pltpu.VMEM((NPT, CHH), jnp.float32),
        pltpu.VMEM((NPT, CHH), jnp.float32),
        pltpu.VMEM((ECH, CHH), jnp.float32),
        pltpu.VMEM((ECH, CHH), jnp.float32),
        pltpu.VMEM((ECH, CHH), jnp.float32),
        pltpu.VMEM((ECH, CHH), jnp.float32),
        pltpu.VMEM((CPB, 2, ECH), jnp.int32),
        pltpu.VMEM((CPB, 2, ECH), jnp.int32),
        pltpu.VMEM((NPT,), jnp.float32),
        pltpu.VMEM((ECH,), jnp.float32),
        pltpu.VMEM((16,), jnp.float32),
    ]
    + [pltpu.SemaphoreType.DMA] * (2 * NBUF + 2),
    compiler_params=pltpu.CompilerParams(
        needs_layout_passes=False, use_tc_tiling_on_sc=False
    ),
)
def _bern_sc(idx_hbm, h_hbm, coef_hbm, ones_hbm, zrow_hbm, zcol_hbm,
             y_out,
             z_sh, g_sh, deg_sh, y_v, h_v, gbuf,
             r0, r1, r2, r3,
             idxblk0, idxblk1, dinv_v, ones_v, coef_v,
             *sems):
    rb = (r0, r1, r2, r3)
    semr = sems[:NBUF]
    semw = sems[NBUF:2 * NBUF]
    semi = sems[2 * NBUF:]
    idxb = (idxblk0, idxblk1)
    c = lax.axis_index("c")
    s = lax.axis_index("s")
    base = pl.multiple_of(s * NPT, NPT)
    zview = z_sh

    def _zero_g_slice():
        for q in range(NPT // ECH):
            pltpu.sync_copy(zrow_hbm, g_sh.at[pl.ds(base + q * ECH, ECH)])

    pltpu.sync_copy(h_hbm.at[c, pl.ds(base, NPT)], h_v)
    pltpu.sync_copy(coef_hbm, coef_v)
    pltpu.sync_copy(ones_hbm, ones_v)
    pltpu.sync_copy(zcol_hbm, dinv_v)
    pltpu.sync_copy(dinv_v, deg_sh.at[pl.ds(base, NPT)])
    plsc.subcore_barrier()

    def _degblk(b, cc):
        pltpu.sync_copy(idx_hbm.at[s, b], idxblk0)

        def _dchunk(j, c2):
            pltpu.sync_copy(ones_v, deg_sh.at[idxblk0.at[j, 0]], add=True)
            return c2

        lax.fori_loop(0, CPB, _dchunk, 0)
        return cc

    lax.fori_loop(0, NBLK, _degblk, 0)
    plsc.subcore_barrier()

    pltpu.sync_copy(deg_sh.at[pl.ds(base, NPT)], dinv_v)

    def _invsqrt(i, carry):
        d = dinv_v[pl.ds(i * 16, 16)]
        xh = d * 0.5
        ib = lax.bitcast_convert_type(d, jnp.int32)
        ib = 0x5F3759DF - lax.shift_right_arithmetic(ib, 1)
        f = lax.bitcast_convert_type(ib, jnp.float32)
        f = f * (1.5 - xh * f * f)
        f = f * (1.5 - xh * f * f)
        f = f * (1.5 - xh * f * f)
        dinv_v[pl.ds(i * 16, 16)] = jnp.where(d > 0.5, f, 0.0)
        return carry

    lax.fori_loop(0, NPT // 16, _invsqrt, 0)

    ak = plsc.load_gather(coef_v, [_splat(K)])

    def _init_node(n, carry):
        dv = plsc.load_gather(dinv_v, [jnp.full((16,), n, jnp.int32)])
        for half in range(2):
            sl = pl.ds(half * 16, 16)
            yv = ak * h_v[n, sl]
            y_v[n, sl] = yv
            gbuf[n, sl] = dv * yv
        return carry

    lax.fori_loop(0, NPT, _init_node, 0)
    pltpu.sync_copy(gbuf, zview.at[pl.ds(base, NPT)])
    _zero_g_slice()
    plsc.subcore_barrier()

    def _edge_sweep():
        gd = [None] * NBUF
        sd = [None] * NBUF
        iw = [None, None]
        pltpu.sync_copy(idx_hbm.at[s, 0], idxblk0)
        for k in range(LEAD):
            bq = k % NBUF
            gd[bq] = pltpu.make_async_copy(
                zview.at[idxblk0.at[k, 0]], rb[bq], semr[bq])
            gd[bq].start()
        for b in range(NBLK):
            cur = idxb[b % 2]
            if b + 1 < NBLK:
                iw[(b + 1) % 2] = pltpu.make_async_copy(
                    idx_hbm.at[s, b + 1], idxb[(b + 1) % 2], semi[(b + 1) % 2])
                iw[(b + 1) % 2].start()
            for q in range(CPB):
                k = b * CPB + q
                kk = k + LEAD
                if kk < NCHUNK:
                    tb, tq = divmod(kk, CPB)
                    nb = kk % NBUF
                    if tb != b and tq == 0:
                        iw[tb % 2].wait()
                        iw[tb % 2] = None
                    if sd[nb] is not None:
                        sd[nb].wait()
                        sd[nb] = None
                    gd[nb] = pltpu.make_async_copy(
                        zview.at[idxb[tb % 2].at[tq, 0]], rb[nb], semr[nb])
                    gd[nb].start()
                bq = k % NBUF
                gd[bq].wait()
                gd[bq] = None
                sd[bq] = pltpu.make_async_copy(
                    rb[bq], g_sh.at[cur.at[q, 1]], semw[bq])
                sd[bq].start(add=True)
        for i in range(NBUF):
            if sd[i] is not None:
                sd[i].wait()

    def _horner(t, carry):
        _edge_sweep()
        plsc.subcore_barrier()

        aj = plsc.load_gather(coef_v, [_splat(0) + (K - 1 - t)])
        pltpu.sync_copy(g_sh.at[pl.ds(base, NPT)], gbuf)
        _zero_g_slice()

        def _comb(n, cc):
            dv = plsc.load_gather(dinv_v, [jnp.full((16,), n, jnp.int32)])
            for half in range(2):
                sl = pl.ds(half * 16, 16)
                yv = y_v[n, sl] - dv * gbuf[n, sl] + aj * h_v[n, sl]
                y_v[n, sl] = yv
                gbuf[n, sl] = dv * yv
            return cc

        lax.fori_loop(0, NPT, _comb, 0)
        pltpu.sync_copy(gbuf, zview.at[pl.ds(base, NPT)])
        plsc.subcore_barrier()
        return carry

    lax.fori_loop(0, K, _horner, 0)
    pltpu.sync_copy(y_v, y_out.at[c, pl.ds(base, NPT)])


def kernel(x, edge_index, W0, b0, W1, b1, temp):
    h = _mlp(x, W0, b0.reshape(1, -1), W1, b1.reshape(1, -1))

    a = jnp.sum(jnp.asarray(_CM) * jax.nn.relu(temp)[None, :], axis=1)
    coef = jnp.zeros((16,), jnp.float32).at[: K + 1].set(a)

    h3 = jnp.pad(h, ((0, NP - N), (0, 0))).reshape(NP, NC, CHH).transpose(1, 0, 2)
    row = edge_index[0].astype(jnp.int32)
    col = edge_index[1].astype(jnp.int32)
    pad = jnp.full((EP - E,), N, jnp.int32)
    rowp = jnp.concatenate([row, pad]).reshape(NS, NBLK, CPB, 1, ECH)
    colp = jnp.concatenate([col, pad]).reshape(NS, NBLK, CPB, 1, ECH)
    idx_all = jnp.concatenate([rowp, colp], axis=3)

    y3 = _bern_sc(
        idx_all, h3, coef,
        jnp.ones((ECH,), jnp.float32),
        jnp.zeros((ECH, CHH), jnp.float32),
        jnp.zeros((NPT,), jnp.float32),
    )
    y = y3.transpose(1, 0, 2).reshape(NP, CH)[:N]
    return _lsm(y)


# speedup vs baseline: 2.0043x; 1.0008x over previous
import functools
from math import comb

import jax
import jax.numpy as jnp
import numpy as np
from jax import lax
from jax.experimental import pallas as pl
from jax.experimental.pallas import tpu as pltpu
from jax.experimental.pallas import tpu_sc as plsc

N = 10000
E = 320000
K = 10
CH = 64
NC, NS = 2, 16
CHH = CH // NC
NP = 10240
NPT = NP // NS
ECH = 128
EPT = 20480
NCHUNK = EPT // ECH
CPB = 16
NBLK = NCHUNK // CPB
EP = NS * EPT
RB = 1000

_CM = np.zeros((K + 1, K + 1), np.float64)
for _j in range(K + 1):
    for _i in range(_j + 1):
        _CM[_j, _i] = (2.0 ** -_j) * comb(K, _j) * ((-1) ** (_j - _i)) * comb(_j, _i)
_CM = _CM.astype(np.float32)


def _mlp_body(x_ref, w0_ref, b0_ref, w1_ref, b1_ref, o_ref):
    hh = jnp.dot(
        x_ref[...], w0_ref[...],
        preferred_element_type=jnp.float32, precision=lax.Precision.HIGHEST,
    )
    hh = jnp.maximum(hh + b0_ref[...], 0.0)
    o_ref[...] = (
        jnp.dot(
            hh, w1_ref[...],
            preferred_element_type=jnp.float32, precision=lax.Precision.HIGHEST,
        )
        + b1_ref[...]
    )


_mlp = pl.pallas_call(
    _mlp_body,
    grid=(N // RB,),
    in_specs=[
        pl.BlockSpec((RB, 128), lambda i: (i, 0)),
        pl.BlockSpec((128, 128), lambda i: (0, 0)),
        pl.BlockSpec((1, 128), lambda i: (0, 0)),
        pl.BlockSpec((128, CH), lambda i: (0, 0)),
        pl.BlockSpec((1, CH), lambda i: (0, 0)),
    ],
    out_specs=pl.BlockSpec((RB, CH), lambda i: (i, 0)),
    out_shape=jax.ShapeDtypeStruct((N, CH), jnp.float32),
)


def _lsm_body(y_ref, o_ref):
    y = y_ref[...]
    m = jnp.max(y, axis=1, keepdims=True)
    sh = y - m
    ssum = jnp.sum(jnp.exp(sh), axis=1, keepdims=True)
    o_ref[...] = sh - jnp.log(ssum)


_lsm = pl.pallas_call(
    _lsm_body,
    grid=(N // RB,),
    in_specs=[pl.BlockSpec((RB, CH), lambda i: (i, 0))],
    out_specs=pl.BlockSpec((RB, CH), lambda i: (i, 0)),
    out_shape=jax.ShapeDtypeStruct((N, CH), jnp.float32),
)


_mesh = plsc.VectorSubcoreMesh(
    core_axis_name="c", subcore_axis_name="s", num_cores=NC, num_subcores=NS
)


def _splat(val):
    return jnp.full((16,), val, jnp.int32)


NBUF = 4
LEAD = 2


@functools.partial(
    pl.kernel,
    out_type=jax.ShapeDtypeStruct((NC, NP, CHH), jnp.float32),
    mesh=_mesh,
    scratch_types=[
        pltpu.VMEM_SHARED((NP, CHH), jnp.float32),
        pltpu.VMEM_SHARED((NP, CHH), jnp.float32),
        pltpu.VMEM_SHARED((NP,), jnp.float32),
        pltpu.VMEM((NPT, CHH), jnp.float32),
        pltpu.VMEM((NPT, CHH), jnp.float32),
        pltpu.VMEM((NPT, CHH), jnp.float32),
        pltpu.VMEM((ECH, CHH), jnp.float32),
        pltpu.VMEM((ECH, CHH), jnp.float32),
        pltpu.VMEM((ECH, CHH), jnp.float32),
        pltpu.VMEM((ECH, CHH), jnp.float32),
        pltpu.VMEM((CPB, 2, ECH), jnp.int32),
        pltpu.VMEM((CPB, 2, ECH), jnp.int32),
        pltpu.VMEM((NPT,), jnp.float32),
        pltpu.VMEM((ECH,), jnp.float32),
        pltpu.VMEM((16,), jnp.float32),
    ]
    + [pltpu.SemaphoreType.DMA] * (2 * NBUF + 2),
    compiler_params=pltpu.CompilerParams(
        needs_layout_passes=False, use_tc_tiling_on_sc=False
    ),
)
def _bern_sc(idx_hbm, h_hbm, coef_hbm, ones_hbm, zrow_hbm, zcol_hbm,
             y_out,
             z_sh, g_sh, deg_sh, y_v, h_v, gbuf,
             r0, r1, r2, r3,
             idxblk0, idxblk1, dinv_v, ones_v, coef_v,
             *sems):
    rb = (r0, r1, r2, r3)
    semr = sems[:NBUF]
    semw = sems[NBUF:2 * NBUF]
    semi = sems[2 * NBUF:]
    idxb = (idxblk0, idxblk1)
    c = lax.axis_index("c")
    s = lax.axis_index("s")
    base = pl.multiple_of(s * NPT, NPT)
    zview = z_sh

    def _zero_g_slice():
        for q in range(NPT // ECH):
            pltpu.sync_copy(zrow_hbm, g_sh.at[pl.ds(base + q * ECH, ECH)])

    pltpu.sync_copy(h_hbm.at[c, pl.ds(base, NPT)], h_v)
    pltpu.sync_copy(coef_hbm, coef_v)
    pltpu.sync_copy(ones_hbm, ones_v)
    pltpu.sync_copy(zcol_hbm, dinv_v)
    pltpu.sync_copy(dinv_v, deg_sh.at[pl.ds(base, NPT)])
    plsc.subcore_barrier()

    def _degblk(b, cc):
        pltpu.sync_copy(idx_hbm.at[s, b], idxblk0)

        def _dchunk(j, c2):
            pltpu.sync_copy(ones_v, deg_sh.at[idxblk0.at[j, 0]], add=True)
            return c2

        lax.fori_loop(0, CPB, _dchunk, 0)
        return cc

    lax.fori_loop(0, NBLK, _degblk, 0)
    plsc.subcore_barrier()

    pltpu.sync_copy(deg_sh.at[pl.ds(base, NPT)], dinv_v)

    def _invsqrt(i, carry):
        d = dinv_v[pl.ds(i * 16, 16)]
        xh = d * 0.5
        ib = lax.bitcast_convert_type(d, jnp.int32)
        ib = 0x5F3759DF - lax.shift_right_arithmetic(ib, 1)
        f = lax.bitcast_convert_type(ib, jnp.float32)
        f = f * (1.5 - xh * f * f)
        f = f * (1.5 - xh * f * f)
        f = f * (1.5 - xh * f * f)
        dinv_v[pl.ds(i * 16, 16)] = jnp.where(d > 0.5, f, 0.0)
        return carry

    lax.fori_loop(0, NPT // 16, _invsqrt, 0)

    ak = plsc.load_gather(coef_v, [_splat(K)])

    def _init_node(n, carry):
        dv = plsc.load_gather(dinv_v, [jnp.full((16,), n, jnp.int32)])
        for half in range(2):
            sl = pl.ds(half * 16, 16)
            yv = ak * h_v[n, sl]
            y_v[n, sl] = yv
            gbuf[n, sl] = dv * yv
        return carry

    lax.fori_loop(0, NPT, _init_node, 0)
    pltpu.sync_copy(gbuf, zview.at[pl.ds(base, NPT)])
    _zero_g_slice()
    plsc.subcore_barrier()

    def _edge_sweep():
        gd = [None] * NBUF
        sd = [None] * NBUF
        iw = [None, None]
        pltpu.sync_copy(idx_hbm.at[s, 0], idxblk0)
        for k in range(LEAD):
            bq = k % NBUF
            gd[bq] = pltpu.make_async_copy(
                zview.at[idxblk0.at[k, 0]], rb[bq], semr[bq])
            gd[bq].start()
        for b in range(NBLK):
            cur = idxb[b % 2]
            if b + 1 < NBLK:
                iw[(b + 1) % 2] = pltpu.make_async_copy(
                    idx_hbm.at[s, b + 1], idxb[(b + 1) % 2], semi[(b + 1) % 2])
                iw[(b + 1) % 2].start()
            for q in range(CPB):
                k = b * CPB + q
                kk = k + LEAD
                if kk < NCHUNK:
                    tb, tq = divmod(kk, CPB)
                    nb = kk % NBUF
                    if tb != b and tq == 0:
                        iw[tb % 2].wait()
                        iw[tb % 2] = None
                    if sd[nb] is not None:
                        sd[nb].wait()
                        sd[nb] = None
                    gd[nb] = pltpu.make_async_copy(
                        zview.at[idxb[tb % 2].at[tq, 0]], rb[nb], semr[nb])
                    gd[nb].start()
                bq = k % NBUF
                gd[bq].wait()
                gd[bq] = None
                sd[bq] = pltpu.make_async_copy(
                    rb[bq], g_sh.at[cur.at[q, 1]], semw[bq])
                sd[bq].start(add=True)
        for i in range(NBUF):
            if sd[i] is not None:
                sd[i].wait()

    def _horner(t, carry):
        with jax.named_scope("edge_sweep"):
            _edge_sweep()
        plsc.subcore_barrier()

        with jax.named_scope("combine"):
            aj = plsc.load_gather(coef_v, [_splat(0) + (K - 1 - t)])
            pltpu.sync_copy(g_sh.at[pl.ds(base, NPT)], gbuf)
            _zero_g_slice()

            def _comb(n, cc):
                dv = plsc.load_gather(dinv_v, [jnp.full((16,), n, jnp.int32)])
                for half in range(2):
                    sl = pl.ds(half * 16, 16)
                    yv = y_v[n, sl] - dv * gbuf[n, sl] + aj * h_v[n, sl]
                    y_v[n, sl] = yv
                    gbuf[n, sl] = dv * yv
                return cc

            lax.fori_loop(0, NPT, _comb, 0)
            pltpu.sync_copy(gbuf, zview.at[pl.ds(base, NPT)])
        plsc.subcore_barrier()
        return carry

    lax.fori_loop(0, K, _horner, 0)
    pltpu.sync_copy(y_v, y_out.at[c, pl.ds(base, NPT)])


def kernel(x, edge_index, W0, b0, W1, b1, temp):
    h = _mlp(x, W0, b0.reshape(1, -1), W1, b1.reshape(1, -1))

    a = jnp.sum(jnp.asarray(_CM) * jax.nn.relu(temp)[None, :], axis=1)
    coef = jnp.zeros((16,), jnp.float32).at[: K + 1].set(a)

    h3 = jnp.pad(h, ((0, NP - N), (0, 0))).reshape(NP, NC, CHH).transpose(1, 0, 2)
    row = edge_index[0].astype(jnp.int32)
    col = edge_index[1].astype(jnp.int32)
    pad = jnp.full((EP - E,), N, jnp.int32)
    rowp = jnp.concatenate([row, pad]).reshape(NS, NBLK, CPB, 1, ECH)
    colp = jnp.concatenate([col, pad]).reshape(NS, NBLK, CPB, 1, ECH)
    idx_all = jnp.concatenate([rowp, colp], axis=3)

    y3 = _bern_sc(
        idx_all, h3, coef,
        jnp.ones((ECH,), jnp.float32),
        jnp.zeros((ECH, CHH), jnp.float32),
        jnp.zeros((NPT,), jnp.float32),
    )
    y = y3.transpose(1, 0, 2).reshape(NP, CH)[:N]
    return _lsm(y)


# speedup vs baseline: 2.0658x; 1.0307x over previous
import functools
from math import comb

import jax
import jax.numpy as jnp
import numpy as np
from jax import lax
from jax.experimental import pallas as pl
from jax.experimental.pallas import tpu as pltpu
from jax.experimental.pallas import tpu_sc as plsc

N = 10000
E = 320000
K = 10
CH = 64
NC, NS = 2, 16
CHH = CH // NC
NP = 10240
NPT = NP // NS
ECH = 128
EPT = 20480
NCHUNK = EPT // ECH
CPB = 16
NBLK = NCHUNK // CPB
EP = NS * EPT
RB = 1000

_CM = np.zeros((K + 1, K + 1), np.float64)
for _j in range(K + 1):
    for _i in range(_j + 1):
        _CM[_j, _i] = (2.0 ** -_j) * comb(K, _j) * ((-1) ** (_j - _i)) * comb(_j, _i)
_CM = _CM.astype(np.float32)


RBM = 1024


def _mlp_body(x_ref, w0_ref, b0_ref, w1_ref, b1_ref, o_ref):
    hh = jnp.dot(
        x_ref[...], w0_ref[...],
        preferred_element_type=jnp.float32, precision=lax.Precision.HIGHEST,
    )
    hh = jnp.maximum(hh + b0_ref[...], 0.0)
    hh = (
        jnp.dot(
            hh, w1_ref[...],
            preferred_element_type=jnp.float32, precision=lax.Precision.HIGHEST,
        )
        + b1_ref[...]
    )
    o_ref[0] = hh[:, :CHH]
    o_ref[1] = hh[:, CHH:]


_mlp = pl.pallas_call(
    _mlp_body,
    grid=(NP // RBM,),
    in_specs=[
        pl.BlockSpec((RBM, 128), lambda i: (i, 0)),
        pl.BlockSpec((128, 128), lambda i: (0, 0)),
        pl.BlockSpec((1, 128), lambda i: (0, 0)),
        pl.BlockSpec((128, CH), lambda i: (0, 0)),
        pl.BlockSpec((1, CH), lambda i: (0, 0)),
    ],
    out_specs=pl.BlockSpec((NC, RBM, CHH), lambda i: (0, i, 0)),
    out_shape=jax.ShapeDtypeStruct((NC, NP, CHH), jnp.float32),
)


def _lsm_body(y_ref, o_ref):
    y = jnp.concatenate([y_ref[0], y_ref[1]], axis=1)
    m = jnp.max(y, axis=1, keepdims=True)
    sh = y - m
    ssum = jnp.sum(jnp.exp(sh), axis=1, keepdims=True)
    o_ref[...] = sh - jnp.log(ssum)


_lsm = pl.pallas_call(
    _lsm_body,
    grid=(N // RB,),
    in_specs=[pl.BlockSpec((NC, RB, CHH), lambda i: (0, i, 0))],
    out_specs=pl.BlockSpec((RB, CH), lambda i: (i, 0)),
    out_shape=jax.ShapeDtypeStruct((N, CH), jnp.float32),
)


_mesh = plsc.VectorSubcoreMesh(
    core_axis_name="c", subcore_axis_name="s", num_cores=NC, num_subcores=NS
)


def _splat(val):
    return jnp.full((16,), val, jnp.int32)


NBUF = 4
LEAD = 2


@functools.partial(
    pl.kernel,
    out_type=jax.ShapeDtypeStruct((NC, NP, CHH), jnp.float32),
    mesh=_mesh,
    scratch_types=[
        pltpu.VMEM_SHARED((NP, CHH), jnp.float32),
        pltpu.VMEM_SHARED((NP, CHH), jnp.float32),
        pltpu.VMEM_SHARED((NP,), jnp.float32),
        pltpu.VMEM((NPT, CHH), jnp.float32),
        pltpu.VMEM((NPT, CHH), jnp.float32),
        pltpu.VMEM((NPT, CHH), jnp.float32),
        pltpu.VMEM((ECH, CHH), jnp.float32),
        pltpu.VMEM((ECH, CHH), jnp.float32),
        pltpu.VMEM((ECH, CHH), jnp.float32),
        pltpu.VMEM((ECH, CHH), jnp.float32),
        pltpu.VMEM((CPB, 2, ECH), jnp.int32),
        pltpu.VMEM((CPB, 2, ECH), jnp.int32),
        pltpu.VMEM((NPT,), jnp.float32),
        pltpu.VMEM((ECH,), jnp.float32),
        pltpu.VMEM((16,), jnp.float32),
    ]
    + [pltpu.SemaphoreType.DMA] * (2 * NBUF + 2),
    compiler_params=pltpu.CompilerParams(
        needs_layout_passes=False, use_tc_tiling_on_sc=False
    ),
)
def _bern_sc(idx_hbm, h_hbm, coef_hbm, ones_hbm, zrow_hbm, zcol_hbm,
             y_out,
             z_sh, g_sh, deg_sh, y_v, h_v, gbuf,
             r0, r1, r2, r3,
             idxblk0, idxblk1, dinv_v, ones_v, coef_v,
             *sems):
    rb = (r0, r1, r2, r3)
    semr = sems[:NBUF]
    semw = sems[NBUF:2 * NBUF]
    semi = sems[2 * NBUF:]
    idxb = (idxblk0, idxblk1)
    c = lax.axis_index("c")
    s = lax.axis_index("s")
    base = pl.multiple_of(s * NPT, NPT)
    zview = z_sh

    def _zero_g_slice():
        for q in range(NPT // ECH):
            pltpu.sync_copy(zrow_hbm, g_sh.at[pl.ds(base + q * ECH, ECH)])

    pltpu.sync_copy(h_hbm.at[c, pl.ds(base, NPT)], h_v)
    pltpu.sync_copy(coef_hbm, coef_v)
    pltpu.sync_copy(ones_hbm, ones_v)
    pltpu.sync_copy(zcol_hbm, dinv_v)
    pltpu.sync_copy(dinv_v, deg_sh.at[pl.ds(base, NPT)])
    plsc.subcore_barrier()

    def _degblk(b, cc):
        pltpu.sync_copy(idx_hbm.at[s, b], idxblk0)

        def _dchunk(j, c2):
            pltpu.sync_copy(ones_v, deg_sh.at[idxblk0.at[j, 0]], add=True)
            return c2

        lax.fori_loop(0, CPB, _dchunk, 0)
        return cc

    lax.fori_loop(0, NBLK, _degblk, 0)
    plsc.subcore_barrier()

    pltpu.sync_copy(deg_sh.at[pl.ds(base, NPT)], dinv_v)

    def _invsqrt(i, carry):
        d = dinv_v[pl.ds(i * 16, 16)]
        xh = d * 0.5
        ib = lax.bitcast_convert_type(d, jnp.int32)
        ib = 0x5F3759DF - lax.shift_right_arithmetic(ib, 1)
        f = lax.bitcast_convert_type(ib, jnp.float32)
        f = f * (1.5 - xh * f * f)
        f = f * (1.5 - xh * f * f)
        f = f * (1.5 - xh * f * f)
        dinv_v[pl.ds(i * 16, 16)] = jnp.where(d > 0.5, f, 0.0)
        return carry

    lax.fori_loop(0, NPT // 16, _invsqrt, 0)

    ak = plsc.load_gather(coef_v, [_splat(K)])

    def _init_node(i, carry):
        for u in range(4):
            n = i * 4 + u
            dv = plsc.load_gather(dinv_v, [jnp.full((16,), n, jnp.int32)])
            for half in range(2):
                sl = pl.ds(half * 16, 16)
                yv = ak * h_v[n, sl]
                y_v[n, sl] = yv
                gbuf[n, sl] = dv * yv
        return carry

    lax.fori_loop(0, NPT // 4, _init_node, 0)
    pltpu.sync_copy(gbuf, zview.at[pl.ds(base, NPT)])
    _zero_g_slice()
    plsc.subcore_barrier()

    def _edge_sweep():
        gd = [None] * NBUF
        sd = [None] * NBUF
        iw = [None, None]
        pltpu.sync_copy(idx_hbm.at[s, 0], idxblk0)
        for k in range(LEAD):
            bq = k % NBUF
            gd[bq] = pltpu.make_async_copy(
                zview.at[idxblk0.at[k, 0]], rb[bq], semr[bq])
            gd[bq].start()
        for b in range(NBLK):
            cur = idxb[b % 2]
            if b + 1 < NBLK:
                iw[(b + 1) % 2] = pltpu.make_async_copy(
                    idx_hbm.at[s, b + 1], idxb[(b + 1) % 2], semi[(b + 1) % 2])
                iw[(b + 1) % 2].start()
            for q in range(CPB):
                k = b * CPB + q
                kk = k + LEAD
                if kk < NCHUNK:
                    tb, tq = divmod(kk, CPB)
                    nb = kk % NBUF
                    if tb != b and tq == 0:
                        iw[tb % 2].wait()
                        iw[tb % 2] = None
                    if sd[nb] is not None:
                        sd[nb].wait()
                        sd[nb] = None
                    gd[nb] = pltpu.make_async_copy(
                        zview.at[idxb[tb % 2].at[tq, 0]], rb[nb], semr[nb])
                    gd[nb].start()
                bq = k % NBUF
                gd[bq].wait()
                gd[bq] = None
                sd[bq] = pltpu.make_async_copy(
                    rb[bq], g_sh.at[cur.at[q, 1]], semw[bq])
                sd[bq].start(add=True)
        for i in range(NBUF):
            if sd[i] is not None:
                sd[i].wait()

    def _horner(t, carry):
        with jax.named_scope("edge_sweep"):
            _edge_sweep()
        plsc.subcore_barrier()

        with jax.named_scope("combine"):
            aj = plsc.load_gather(coef_v, [_splat(0) + (K - 1 - t)])
            pltpu.sync_copy(g_sh.at[pl.ds(base, NPT)], gbuf)
            _zero_g_slice()

            def _comb(i, cc):
                for u in range(4):
                    n = i * 4 + u
                    dv = plsc.load_gather(dinv_v, [jnp.full((16,), n, jnp.int32)])
                    for half in range(2):
                        sl = pl.ds(half * 16, 16)
                        yv = y_v[n, sl] - dv * gbuf[n, sl] + aj * h_v[n, sl]
                        y_v[n, sl] = yv
                        gbuf[n, sl] = dv * yv
                return cc

            lax.fori_loop(0, NPT // 4, _comb, 0)
            pltpu.sync_copy(gbuf, zview.at[pl.ds(base, NPT)])
        plsc.subcore_barrier()
        return carry

    lax.fori_loop(0, K, _horner, 0)
    pltpu.sync_copy(y_v, y_out.at[c, pl.ds(base, NPT)])


def kernel(x, edge_index, W0, b0, W1, b1, temp):
    h3 = _mlp(x, W0, b0.reshape(1, -1), W1, b1.reshape(1, -1))

    a = jnp.sum(jnp.asarray(_CM) * jax.nn.relu(temp)[None, :], axis=1)
    coef = jnp.zeros((16,), jnp.float32).at[: K + 1].set(a)

    row = edge_index[0].astype(jnp.int32)
    col = edge_index[1].astype(jnp.int32)
    pad = jnp.full((EP - E,), N, jnp.int32)
    rowp = jnp.concatenate([row, pad]).reshape(NS, NBLK, CPB, 1, ECH)
    colp = jnp.concatenate([col, pad]).reshape(NS, NBLK, CPB, 1, ECH)
    idx_all = jnp.concatenate([rowp, colp], axis=3)

    y3 = _bern_sc(
        idx_all, h3, coef,
        jnp.ones((ECH,), jnp.float32),
        jnp.zeros((ECH, CHH), jnp.float32),
        jnp.zeros((NPT,), jnp.float32),
    )
    return _lsm(y3)
